# Initial kernel scaffold; baseline (speedup 1.0000x reference)
import jax
import jax.numpy as jnp
from jax.experimental import pallas as pl


def kernel(x_dense, x_sparse, W_lin, b_lin, table, W1, b1, W2, b2, W3, b3, W4, b4):
    raise NotImplementedError("write your pallas kernel here")


# speedup vs baseline: 7.1594x; 7.1594x over previous
import functools

import jax
import jax.numpy as jnp
from jax import lax
from jax.experimental import pallas as pl
from jax.experimental.pallas import tpu as pltpu
from jax.experimental.pallas import tpu_sc as plsc

NUM_FIELDS = 26
FIELD_SIZE = 100000
EMBED_DIM = 16
BATCH = 16384

NC, NS, L = 2, 16, 16
NW = NC * NS
N_ROWS = BATCH * NUM_FIELDS
ROWS_PER_W = N_ROWS // NW
CHUNK = 1664
N_CHUNKS = ROWS_PER_W // CHUNK

@functools.cache
def _make_sc_gather():
    mesh = plsc.VectorSubcoreMesh(
        core_axis_name="c", subcore_axis_name="s",
        num_cores=NC, num_subcores=NS)

    @functools.partial(
        pl.kernel,
        out_type=jax.ShapeDtypeStruct((N_ROWS, EMBED_DIM), jnp.float32),
        mesh=mesh,
        scratch_types=[
            pltpu.VMEM((CHUNK,), jnp.int32),
            pltpu.VMEM((CHUNK, EMBED_DIM), jnp.float32),
            pltpu.SemaphoreType.DMA,
        ],
        compiler_params=pltpu.CompilerParams(use_tc_tiling_on_sc=False),
    )
    def _sc_gather(xs_hbm, table_hbm, out_hbm, idx_v, rows_v, sem):
        wid = lax.axis_index("s") * NC + lax.axis_index("c")
        base = wid * ROWS_PER_W

        def body(ci, _):
            cbase = base + ci * CHUNK
            pltpu.sync_copy(xs_hbm.at[pl.ds(cbase, CHUNK)], idx_v)
            for v in range(CHUNK // L):
                pos = cbase + v * L + lax.broadcasted_iota(jnp.int32, (L,), 0)
                off = (pos % NUM_FIELDS) * FIELD_SIZE
                idx_v[pl.ds(v * L, L)] = idx_v[pl.ds(v * L, L)] + off
            pltpu.async_copy(table_hbm.at[idx_v], rows_v, sem).wait()
            pltpu.sync_copy(rows_v, out_hbm.at[pl.ds(cbase, CHUNK)])
            return 0

        lax.fori_loop(0, N_CHUNKS, body, 0)

    return _sc_gather


BB = 1024


def _mlp_body(emb_ref, xd_ref, wlin_ref, w1_ref, b1_ref, w2_ref, b2_ref,
              w3_ref, b3_ref, w4_ref, bias_ref, out_ref):
    h = emb_ref[...]
    h = jnp.maximum(
        jnp.dot(h, w1_ref[...], preferred_element_type=jnp.float32)
        + b1_ref[...], 0.0)
    h = jnp.maximum(
        jnp.dot(h, w2_ref[...], preferred_element_type=jnp.float32)
        + b2_ref[...], 0.0)
    h = jnp.maximum(
        jnp.dot(h, w3_ref[...], preferred_element_type=jnp.float32)
        + b3_ref[...], 0.0)
    y_deep = jnp.sum(h * w4_ref[...], axis=1, keepdims=True)
    y_wide = jnp.sum(xd_ref[...] * wlin_ref[...], axis=1, keepdims=True)
    out_ref[...] = jax.nn.sigmoid(y_deep + y_wide + bias_ref[0, 0])


def _mlp(emb, x_dense, wlin_row, W1, b1, W2, b2, W3, b3, w4_row, bias):
    nb = BATCH // BB
    full = lambda shape: pl.BlockSpec(shape, lambda i: (0, 0))
    return pl.pallas_call(
        _mlp_body,
        grid=(nb,),
        in_specs=[
            pl.BlockSpec((BB, NUM_FIELDS * EMBED_DIM), lambda i: (i, 0)),
            pl.BlockSpec((BB, 13), lambda i: (i, 0)),
            full((1, 13)),
            full(W1.shape),
            full((1, 1024)),
            full(W2.shape),
            full((1, 512)),
            full(W3.shape),
            full((1, 256)),
            full((1, 256)),
            full((1, 1)),
        ],
        out_specs=pl.BlockSpec((BB, 1), lambda i: (i, 0)),
        out_shape=jax.ShapeDtypeStruct((BATCH, 1), jnp.float32),
    )(emb, x_dense, wlin_row, W1, b1, W2, b2, W3, b3, w4_row, bias)


def kernel(x_dense, x_sparse, W_lin, b_lin, table, W1, b1, W2, b2, W3, b3,
           W4, b4):
    xs_flat = x_sparse.astype(jnp.int32).reshape(-1)
    emb = _make_sc_gather()(xs_flat, table)
    emb = emb.reshape(BATCH, NUM_FIELDS * EMBED_DIM)
    bias = (b_lin + b4).reshape(1, 1)
    y = _mlp(emb, x_dense, W_lin.reshape(1, 13), W1, b1.reshape(1, 1024),
             W2, b2.reshape(1, 512), W3, b3.reshape(1, 256),
             W4.reshape(1, 256), bias)
    return y[:, 0]
